# Initial kernel scaffold; baseline (speedup 1.0000x reference)
import jax
import jax.numpy as jnp
from jax.experimental import pallas as pl


def kernel(p_loc, p_conf, p_landms, anchors):
    raise NotImplementedError("write your pallas kernel here")


# speedup vs baseline: 34.3806x; 34.3806x over previous
import functools

import jax
import jax.numpy as jnp
from jax import lax
from jax.experimental import pallas as pl
from jax.experimental.pallas import tpu as pltpu

N = 5000
NPAD = 5120
K = 2048
PRE = 2000
OUT = 384
VAR_XY = 0.1
VAR_WH = 0.2
THR_NMS = 0.3
RB = 256
CB = 256
IB = 128

_HI = jax.lax.Precision.HIGHEST


def _body(loc_ref, key_ref, lm_ref, anch_ref, keyr_ref,
          out_ref, data_ref, cc_ref, ct_ref, m_ref):
    f32 = jnp.float32

    axy = anch_ref[:, 0:2]
    awh = anch_ref[:, 2:4]
    loc = loc_ref[...]
    xy = axy + loc[:, 0:2] * VAR_XY * awh
    wh = awh * jnp.exp(loc[:, 2:4] * VAR_WH)
    half = wh * 0.5
    ltrb = jnp.concatenate([xy - half, xy + half], axis=1)
    axy5 = jnp.concatenate([axy, axy, axy, axy, axy], axis=1)
    awh5 = jnp.concatenate([awh, awh, awh, awh, awh], axis=1)
    kps = axy5 + lm_ref[...] * VAR_XY * awh5
    key = key_ref[...]
    sig = jax.nn.sigmoid(key)
    sigr = jax.nn.sigmoid(keyr_ref[...])
    data_ref[...] = jnp.concatenate([ltrb, kps, sig, sig], axis=1)

    i32 = jnp.int32
    jdx = lax.broadcasted_iota(i32, (1, NPAD), 1)

    def rank_step(ib, acc):
        sb = jax.nn.sigmoid(key_ref[pl.ds(ib * RB, RB), :])
        ii = lax.broadcasted_iota(i32, (RB, 1), 0) + ib * RB
        beats = (sb > sigr) | ((sb == sigr) & (ii < jdx))
        return acc + jnp.sum(beats.astype(f32), axis=0, keepdims=True)

    rank = lax.fori_loop(0, NPAD // RB, rank_step,
                         jnp.zeros((1, NPAD), f32))

    def comp_step(rb, carry):
        rcol = (lax.broadcasted_iota(i32, (CB, 1), 0)
                + rb * CB).astype(f32)
        oh = (rank == rcol).astype(f32)
        cc_ref[pl.ds(rb * CB, CB), :] = jnp.dot(
            oh, data_ref[...], precision=_HI, preferred_element_type=f32)
        return carry

    lax.fori_loop(0, K // CB, comp_step, 0)
    ct_ref[...] = jnp.transpose(cc_ref[...])

    x1r = ct_ref[0:1, :]
    y1r = ct_ref[1:2, :]
    x2r = ct_ref[2:3, :]
    y2r = ct_ref[3:4, :]
    arear = jnp.maximum(x2r - x1r, 0.0) * jnp.maximum(y2r - y1r, 0.0)
    jidx = lax.broadcasted_iota(i32, (1, K), 1)

    def iou_step(ib, carry):
        bb = cc_ref[pl.ds(ib * IB, IB), 0:4]
        x1c, y1c, x2c, y2c = (bb[:, 0:1], bb[:, 1:2], bb[:, 2:3], bb[:, 3:4])
        areac = jnp.maximum(x2c - x1c, 0.0) * jnp.maximum(y2c - y1c, 0.0)
        iw = jnp.maximum(jnp.minimum(x2c, x2r) - jnp.maximum(x1c, x1r), 0.0)
        ih = jnp.maximum(jnp.minimum(y2c, y2r) - jnp.maximum(y1c, y1r), 0.0)
        inter = iw * ih
        iou = inter / jnp.maximum(areac + arear - inter, 1e-9)
        iidx = lax.broadcasted_iota(i32, (IB, 1), 0) + ib * IB
        m = (iou > THR_NMS) & (iidx < jidx) & (iidx < PRE) & (jidx < PRE)
        m_ref[pl.ds(ib * IB, IB), :] = m.astype(jnp.bfloat16)
        return carry

    lax.fori_loop(0, K // IB, iou_step, 0)

    in_pre = jidx < PRE
    keep0 = in_pre.astype(f32)

    def cond(c):
        t, _, changed = c
        return changed & (t < K)

    def step(c):
        t, keep, _ = c
        s = jnp.dot(keep.astype(jnp.bfloat16), m_ref[...],
                    preferred_element_type=f32)
        nk = ((s == 0.0) & in_pre).astype(f32)
        changed = jnp.sum(jnp.abs(nk - keep)) > 0.0
        return (t + jnp.int32(1), nk, changed)

    _, keep, _ = lax.while_loop(cond, step, (jnp.int32(0), keep0,
                                             jnp.bool_(True)))

    sct = ct_ref[14:15, :]
    valid = (keep > 0.0) & (sct >= 0.5) & in_pre
    vf = valid.astype(f32)
    x = vf
    s = 1
    while s < K:
        x = x + jnp.concatenate(
            [jnp.zeros((1, s), f32), x[:, :K - s]], axis=1)
        s *= 2
    pos = x - vf

    rcol = lax.broadcasted_iota(i32, (OUT, 1), 0).astype(f32)
    oh = ((pos == rcol) & valid).astype(f32)
    out_ref[...] = jnp.dot(oh, cc_ref[...], precision=_HI,
                           preferred_element_type=f32)


@functools.partial(jax.jit, static_argnums=())
def kernel(p_loc, p_conf, p_landms, anchors):
    pad = NPAD - N
    locp = jnp.pad(p_loc[0], ((0, pad), (0, 0)))
    keyp = jnp.pad(p_conf[0], ((0, pad), (0, 0)), constant_values=-3.4e38)
    lmp = jnp.pad(p_landms[0], ((0, pad), (0, 0)))
    anchp = jnp.pad(anchors, ((0, pad), (0, 0)), constant_values=0.5)
    keyr = keyp.T

    out = pl.pallas_call(
        _body,
        out_shape=jax.ShapeDtypeStruct((OUT, 16), jnp.float32),
        scratch_shapes=[
            pltpu.VMEM((NPAD, 16), jnp.float32),
            pltpu.VMEM((K, 16), jnp.float32),
            pltpu.VMEM((16, K), jnp.float32),
            pltpu.VMEM((K, K), jnp.bfloat16),
        ],
    )(locp, keyp, lmp, anchp, keyr)

    p_boxes = out[:300, 0:4]
    p_keypoints = out[:300, 4:14]
    p_scores = out[:300, 15]
    return (p_boxes, p_keypoints, p_scores)


# speedup vs baseline: 34.7792x; 1.0116x over previous
import functools

import jax
import jax.numpy as jnp
from jax import lax
from jax.experimental import pallas as pl
from jax.experimental.pallas import tpu as pltpu

N = 5000
NPAD = 5120
K = 2048
PRE = 2000
OUT = 384
VAR_XY = 0.1
VAR_WH = 0.2
THR_NMS = 0.3
RB = 256
CB = 256
IB = 128

_HI = jax.lax.Precision.HIGHEST


def _body(loc_ref, key_ref, lm_ref, anch_ref, keyr_ref,
          out_ref, data_ref, cc_ref, ct_ref, m_ref):
    f32 = jnp.float32

    axy = anch_ref[:, 0:2]
    awh = anch_ref[:, 2:4]
    loc = loc_ref[...]
    xy = axy + loc[:, 0:2] * VAR_XY * awh
    wh = awh * jnp.exp(loc[:, 2:4] * VAR_WH)
    half = wh * 0.5
    ltrb = jnp.concatenate([xy - half, xy + half], axis=1)
    axy5 = jnp.concatenate([axy, axy, axy, axy, axy], axis=1)
    awh5 = jnp.concatenate([awh, awh, awh, awh, awh], axis=1)
    kps = axy5 + lm_ref[...] * VAR_XY * awh5
    key = key_ref[...]
    sig = jax.nn.sigmoid(key)
    sigr = jax.nn.sigmoid(keyr_ref[...])
    data_ref[0:N, :] = jnp.concatenate([ltrb, kps, sig, sig], axis=1)
    data_ref[N:NPAD, :] = jnp.zeros((NPAD - N, 16), f32)

    i32 = jnp.int32
    jdx = lax.broadcasted_iota(i32, (1, NPAD), 1)

    def rank_step(ib, acc):
        sb = data_ref[pl.ds(ib * RB, RB), 14:15]
        ii = lax.broadcasted_iota(i32, (RB, 1), 0) + ib * RB
        beats = (sb > sigr) | ((sb == sigr) & (ii < jdx))
        return acc + jnp.sum(beats.astype(f32), axis=0, keepdims=True)

    rank = lax.fori_loop(0, NPAD // RB, rank_step,
                         jnp.zeros((1, NPAD), f32))

    def comp_step(rb, carry):
        rcol = (lax.broadcasted_iota(i32, (CB, 1), 0)
                + rb * CB).astype(f32)
        oh = (rank == rcol).astype(f32)
        cc_ref[pl.ds(rb * CB, CB), :] = jnp.dot(
            oh, data_ref[...], precision=_HI, preferred_element_type=f32)
        return carry

    lax.fori_loop(0, K // CB, comp_step, 0)
    ct_ref[...] = jnp.transpose(cc_ref[...])

    x1r = ct_ref[0:1, :]
    y1r = ct_ref[1:2, :]
    x2r = ct_ref[2:3, :]
    y2r = ct_ref[3:4, :]
    arear = jnp.maximum(x2r - x1r, 0.0) * jnp.maximum(y2r - y1r, 0.0)
    jidx = lax.broadcasted_iota(i32, (1, K), 1)

    def iou_step(ib, carry):
        bb = cc_ref[pl.ds(ib * IB, IB), 0:4]
        x1c, y1c, x2c, y2c = (bb[:, 0:1], bb[:, 1:2], bb[:, 2:3], bb[:, 3:4])
        areac = jnp.maximum(x2c - x1c, 0.0) * jnp.maximum(y2c - y1c, 0.0)
        iw = jnp.maximum(jnp.minimum(x2c, x2r) - jnp.maximum(x1c, x1r), 0.0)
        ih = jnp.maximum(jnp.minimum(y2c, y2r) - jnp.maximum(y1c, y1r), 0.0)
        inter = iw * ih
        iou = inter / jnp.maximum(areac + arear - inter, 1e-9)
        iidx = lax.broadcasted_iota(i32, (IB, 1), 0) + ib * IB
        m = (iou > THR_NMS) & (iidx < jidx) & (iidx < PRE) & (jidx < PRE)
        m_ref[pl.ds(ib * IB, IB), :] = m.astype(jnp.bfloat16)
        return carry

    lax.fori_loop(0, K // IB, iou_step, 0)

    in_pre = jidx < PRE
    keep0 = in_pre.astype(f32)

    def cond(c):
        t, _, changed = c
        return changed & (t < K)

    def step(c):
        t, keep, _ = c
        s = jnp.dot(keep.astype(jnp.bfloat16), m_ref[...],
                    preferred_element_type=f32)
        nk = ((s == 0.0) & in_pre).astype(f32)
        changed = jnp.sum(jnp.abs(nk - keep)) > 0.0
        return (t + jnp.int32(1), nk, changed)

    _, keep, _ = lax.while_loop(cond, step, (jnp.int32(0), keep0,
                                             jnp.bool_(True)))

    sct = ct_ref[14:15, :]
    valid = (keep > 0.0) & (sct >= 0.5) & in_pre
    vf = valid.astype(f32)
    x = vf
    s = 1
    while s < K:
        x = x + jnp.concatenate(
            [jnp.zeros((1, s), f32), x[:, :K - s]], axis=1)
        s *= 2
    pos = x - vf

    rcol = lax.broadcasted_iota(i32, (OUT, 1), 0).astype(f32)
    oh = ((pos == rcol) & valid).astype(f32)
    out_ref[...] = jnp.dot(oh, cc_ref[...], precision=_HI,
                           preferred_element_type=f32)


@functools.partial(jax.jit, static_argnums=())
def kernel(p_loc, p_conf, p_landms, anchors):
    loc = p_loc.reshape(N, 4)
    keyc = p_conf.reshape(N, 1)
    lm = p_landms.reshape(N, 10)
    keyr = jnp.pad(p_conf.reshape(1, N), ((0, 0), (0, NPAD - N)),
                   constant_values=-3.4e38)

    out = pl.pallas_call(
        _body,
        out_shape=jax.ShapeDtypeStruct((OUT, 16), jnp.float32),
        scratch_shapes=[
            pltpu.VMEM((NPAD, 16), jnp.float32),
            pltpu.VMEM((K, 16), jnp.float32),
            pltpu.VMEM((16, K), jnp.float32),
            pltpu.VMEM((K, K), jnp.bfloat16),
        ],
    )(loc, keyc, lm, anchors, keyr)

    p_boxes = out[:300, 0:4]
    p_keypoints = out[:300, 4:14]
    p_scores = out[:300, 15]
    return (p_boxes, p_keypoints, p_scores)
